# baseline (speedup 1.0000x reference)
import functools
import math

import jax
import jax.numpy as jnp
from jax import lax
from jax.experimental import pallas as pl
from jax.experimental.pallas import tpu as pltpu
from jax.experimental.pallas import tpu_sc as plsc

_D = 1024
_L = 16
_NC = 2
_NS = 16
_NW = _NC * _NS
_C = 8
_NBUF = 14
_SCALE = math.sqrt(_D)


@functools.partial(jax.jit, static_argnames=("batch", "seq"))
def _gather_scale(idx, weight, batch, seq):
    n_rows = batch * seq
    rows_per_w = n_rows // _NW
    w_per_b = seq // rows_per_w
    n_chunks = rows_per_w // _C
    mesh = plsc.VectorSubcoreMesh(core_axis_name="c", subcore_axis_name="s")

    @functools.partial(
        pl.kernel,
        out_type=jax.ShapeDtypeStruct((batch, seq, _D), jnp.float32),
        mesh=mesh,
        scratch_types=[
            pltpu.VMEM((rows_per_w,), jnp.int32),
            pltpu.VMEM((_NBUF, _C, _D), jnp.float32),
            pltpu.SemaphoreType.DMA((_NBUF,)),
            pltpu.SemaphoreType.DMA((_NBUF,)),
        ],
    )
    def body(idx_hbm, w_hbm, out_hbm, idx_v, bufs, gsems, ssems):
        wid = lax.axis_index("s") * _NC + lax.axis_index("c")
        b_idx = wid // w_per_b
        s_base = (wid % w_per_b) * rows_per_w
        pltpu.sync_copy(idx_hbm.at[b_idx, pl.ds(s_base, rows_per_w)], idx_v)

        def gather(j, slot):
            pltpu.async_copy(
                w_hbm.at[idx_v.at[pl.ds(j * _C, _C)]],
                bufs.at[slot],
                gsems.at[slot],
            )

        def scatter(j, slot):
            pltpu.async_copy(
                bufs.at[slot],
                out_hbm.at[b_idx, pl.ds(s_base + j * _C, _C)],
                ssems.at[slot],
            )

        def wait_gather(slot):
            pltpu.make_async_copy(
                w_hbm.at[pl.ds(0, _C)], bufs.at[slot], gsems.at[slot]
            ).wait()

        def wait_scatter(slot):
            pltpu.make_async_copy(
                bufs.at[slot], out_hbm.at[0, pl.ds(0, _C)], ssems.at[slot]
            ).wait()

        def scale(slot):
            def row(r, carry):
                for i in range(_D // _L):
                    sl = pl.ds(i * _L, _L)
                    bufs[slot, r, sl] = bufs[slot, r, sl] * _SCALE
                return carry

            lax.fori_loop(0, _C, row, 0)

        for j in range(_NBUF - 1):
            gather(j, j)

        def step(j, slot):
            wait_gather(slot)
            scale(slot)
            scatter(j, slot)
            nxt_slot = lax.select(slot == 0, _NBUF - 1, slot - 1)

            @pl.when(j >= 1)
            def _():
                wait_scatter(nxt_slot)

            @pl.when(j + _NBUF - 1 < n_chunks)
            def _():
                gather(j + _NBUF - 1, nxt_slot)

            return lax.select(slot == _NBUF - 1, 0, slot + 1)

        lax.fori_loop(0, n_chunks, step, 0)
        wait_scatter((n_chunks - 1) % _NBUF)

    return body(idx, weight)


def kernel(input_ids, weight):
    b, s = input_ids.shape
    return _gather_scale(input_ids.astype(jnp.int32), weight, b, s)


# speedup vs baseline: 2.6792x; 2.6792x over previous
import functools
import math

import jax
import jax.numpy as jnp
from jax import lax
from jax.experimental import pallas as pl
from jax.experimental.pallas import tpu as pltpu
from jax.experimental.pallas import tpu_sc as plsc

_D = 1024
_L = 16
_NC = 2
_NS = 16
_NW = _NC * _NS
_C = 16
_NBUF = 7
_SCALE = math.sqrt(_D)


@functools.partial(jax.jit, static_argnames=("batch", "seq"))
def _gather_scale(idx, weight, batch, seq):
    n_rows = batch * seq
    rows_per_w = n_rows // _NW
    w_per_b = seq // rows_per_w
    n_chunks = rows_per_w // _C
    mesh = plsc.VectorSubcoreMesh(core_axis_name="c", subcore_axis_name="s")

    @functools.partial(
        pl.kernel,
        out_type=jax.ShapeDtypeStruct((batch, seq, _D), jnp.float32),
        mesh=mesh,
        scratch_types=[
            pltpu.VMEM((rows_per_w,), jnp.int32),
            pltpu.VMEM((_NBUF, _C, _D), jnp.float32),
            pltpu.SemaphoreType.DMA((_NBUF,)),
            pltpu.SemaphoreType.DMA((_NBUF,)),
        ],
    )
    def body(idx_hbm, w_hbm, out_hbm, idx_v, bufs, gsems, ssems):
        wid = lax.axis_index("s") * _NC + lax.axis_index("c")
        b_idx = wid // w_per_b
        s_base = (wid % w_per_b) * rows_per_w
        pltpu.sync_copy(idx_hbm.at[b_idx, pl.ds(s_base, rows_per_w)], idx_v)

        def gather(j, slot):
            pltpu.async_copy(
                w_hbm.at[idx_v.at[pl.ds(j * _C, _C)]],
                bufs.at[slot],
                gsems.at[slot],
            )

        def scatter(j, slot):
            pltpu.async_copy(
                bufs.at[slot],
                out_hbm.at[b_idx, pl.ds(s_base + j * _C, _C)],
                ssems.at[slot],
            )

        def wait_gather(slot):
            pltpu.make_async_copy(
                w_hbm.at[pl.ds(0, _C)], bufs.at[slot], gsems.at[slot]
            ).wait()

        def wait_scatter(slot):
            pltpu.make_async_copy(
                bufs.at[slot], out_hbm.at[0, pl.ds(0, _C)], ssems.at[slot]
            ).wait()

        def scale(slot):
            def row(r, carry):
                for i in range(_D // _L):
                    sl = pl.ds(i * _L, _L)
                    bufs[slot, r, sl] = bufs[slot, r, sl] * _SCALE
                return carry

            lax.fori_loop(0, _C, row, 0)

        for j in range(_NBUF - 1):
            gather(j, j)

        def step(j, slot):
            wait_gather(slot)
            scale(slot)
            scatter(j, slot)
            nxt_slot = lax.select(slot == 0, _NBUF - 1, slot - 1)

            @pl.when(j >= 1)
            def _():
                wait_scatter(nxt_slot)

            @pl.when(j + _NBUF - 1 < n_chunks)
            def _():
                gather(j + _NBUF - 1, nxt_slot)

            return lax.select(slot == _NBUF - 1, 0, slot + 1)

        lax.fori_loop(0, n_chunks, step, 0)
        wait_scatter((n_chunks - 1) % _NBUF)

    return body(idx, weight)


def kernel(input_ids, weight):
    b, s = input_ids.shape
    return _gather_scale(input_ids.astype(jnp.int32), weight, b, s)
